# baseline (speedup 1.0000x reference)
import functools

import jax
import jax.numpy as jnp
from jax import lax
from jax.experimental import pallas as pl
from jax.experimental.pallas import tpu as pltpu
from jax.experimental.pallas import tpu_sc as plsc

BATCH = 16384
DIM = 32
NUM_CORES = 2
NUM_SUBCORES = 16
NW = NUM_CORES * NUM_SUBCORES
BPW = BATCH // NW
CHUNK = 128
NCHUNK = BPW // CHUNK
LANES = 16


def _gather_body(user_hbm, item_hbm, eug_hbm, eig_hbm, eum_hbm, eim_hbm,
                 eum_out, eim_out, gmf_out,
                 uidx, iidx, ug_v, ig_v, um_v, im_v, sem):
    wid = lax.axis_index("s") * NUM_CORES + lax.axis_index("c")
    base = wid * BPW

    pltpu.sync_copy(user_hbm.at[wid], uidx)
    pltpu.sync_copy(item_hbm.at[wid], iidx)

    copies = []
    for j in range(NCHUNK):
        rows = pl.ds(j * CHUNK, CHUNK)
        copies.append(pltpu.async_copy(eum_hbm.at[uidx.at[j]], um_v.at[rows], sem))
        copies.append(pltpu.async_copy(eim_hbm.at[iidx.at[j]], im_v.at[rows], sem))
        copies.append(pltpu.async_copy(eug_hbm.at[uidx.at[j]], ug_v.at[rows], sem))
        copies.append(pltpu.async_copy(eig_hbm.at[iidx.at[j]], ig_v.at[rows], sem))
    for c in copies:
        c.wait()

    out_rows = pl.ds(base, BPW)
    pltpu.sync_copy(um_v, eum_out.at[out_rows])
    pltpu.sync_copy(im_v, eim_out.at[out_rows])

    def mul_step(t, carry):
        i = t // (DIM // LANES)
        sl = pl.ds((t % (DIM // LANES)) * LANES, LANES)
        ug_v[i, sl] = ug_v[i, sl] * ig_v[i, sl]
        return carry

    lax.fori_loop(0, BPW * (DIM // LANES), mul_step, 0, unroll=8)
    pltpu.sync_copy(ug_v, gmf_out.at[out_rows])


_sc_gather = functools.partial(
    pl.kernel,
    out_type=[
        jax.ShapeDtypeStruct((BATCH, DIM), jnp.float32),
        jax.ShapeDtypeStruct((BATCH, DIM), jnp.float32),
        jax.ShapeDtypeStruct((BATCH, DIM), jnp.float32),
    ],
    mesh=plsc.VectorSubcoreMesh(core_axis_name="c", subcore_axis_name="s"),
    compiler_params=pltpu.CompilerParams(use_tc_tiling_on_sc=False),
    scratch_types=[
        pltpu.VMEM((NCHUNK, CHUNK), jnp.int32),
        pltpu.VMEM((NCHUNK, CHUNK), jnp.int32),
        pltpu.VMEM((BPW, DIM), jnp.float32),
        pltpu.VMEM((BPW, DIM), jnp.float32),
        pltpu.VMEM((BPW, DIM), jnp.float32),
        pltpu.VMEM((BPW, DIM), jnp.float32),
        pltpu.SemaphoreType.DMA,
    ],
)(_gather_body)


def _mlp_body(eum, eim, gmf, w1u, w1i, b1, w2, b2, w3, b3, wpm, wpg, bp, out):
    h = (jnp.dot(eum[...], w1u[...], preferred_element_type=jnp.float32)
         + jnp.dot(eim[...], w1i[...], preferred_element_type=jnp.float32)
         + b1[...])
    h = jnp.maximum(h, 0.0)
    h = jnp.maximum(jnp.dot(h, w2[...], preferred_element_type=jnp.float32) + b2[...], 0.0)
    h = jnp.maximum(jnp.dot(h, w3[...], preferred_element_type=jnp.float32) + b3[...], 0.0)
    p = (jnp.dot(h, wpm[...], preferred_element_type=jnp.float32)
         + jnp.dot(gmf[...], wpg[...], preferred_element_type=jnp.float32)
         + bp[...])
    out[...] = p


def kernel(user, item, embed_user_GMF, embed_item_GMF, embed_user_MLP, embed_item_MLP,
           W1, b1, W2, b2, W3, b3, Wp, bp):
    user_c = user.astype(jnp.int32).reshape(NW, NCHUNK, CHUNK)
    item_c = item.astype(jnp.int32).reshape(NW, NCHUNK, CHUNK)

    eum, eim, gmf = _sc_gather(user_c, item_c,
                               embed_user_GMF, embed_item_GMF,
                               embed_user_MLP, embed_item_MLP)

    w1u = W1[:, :DIM].T
    w1i = W1[:, DIM:].T
    w2 = W2.T
    w3 = W3.T
    wpg = Wp[:, :DIM].T
    wpm = Wp[:, DIM:].T

    TB = 2048
    grid = (BATCH // TB,)
    row_spec = pl.BlockSpec((TB, DIM), lambda i: (i, 0))
    full = lambda s: pl.BlockSpec(s, lambda i: (0,) * len(s))

    pred = pl.pallas_call(
        _mlp_body,
        grid=grid,
        in_specs=[
            row_spec, row_spec, row_spec,
            full(w1u.shape), full(w1i.shape), full((1, 64)),
            full(w2.shape), full((1, 32)),
            full(w3.shape), full((1, 16)),
            full(wpm.shape), full(wpg.shape), full((1, 1)),
        ],
        out_specs=pl.BlockSpec((TB, 1), lambda i: (i, 0)),
        out_shape=jax.ShapeDtypeStruct((BATCH, 1), jnp.float32),
    )(eum, eim, gmf, w1u, w1i, b1.reshape(1, 64), w2, b2.reshape(1, 32),
      w3, b3.reshape(1, 16), wpm, wpg, bp.reshape(1, 1))

    return pred.reshape(-1)
